# Initial kernel scaffold; baseline (speedup 1.0000x reference)
import jax
import jax.numpy as jnp
from jax.experimental import pallas as pl


def kernel(input, weight):
    raise NotImplementedError("write your pallas kernel here")


# speedup vs baseline: 1.3071x; 1.3071x over previous
import functools

import jax
import jax.numpy as jnp
from jax import lax
from jax.experimental import pallas as pl
from jax.experimental.pallas import tpu as pltpu
from jax.experimental.pallas import tpu_sc as plsc

_CHUNK = 128


@functools.partial(jax.jit, static_argnums=(2, 3))
def _gather(weight, idx2d, B, D):
    info = plsc.get_sparse_core_info()
    NW = info.num_cores * info.num_subcores
    b_per_w = B // NW
    n_chunks = b_per_w // _CHUNK
    mesh = plsc.VectorSubcoreMesh(core_axis_name="c", subcore_axis_name="s")

    @functools.partial(
        pl.kernel,
        mesh=mesh,
        compiler_params=pltpu.CompilerParams(use_tc_tiling_on_sc=False),
        out_type=jax.ShapeDtypeStruct((B, D), jnp.float32),
        scratch_types=[
            pltpu.VMEM((n_chunks, _CHUNK), jnp.int32),
            pltpu.VMEM((_CHUNK, D), jnp.float32),
            pltpu.SemaphoreType.DMA,
        ],
    )
    def k(table_hbm, idx_hbm, out_hbm, idx_v, rows_v, sem):
        wid = lax.axis_index("s") * info.num_cores + lax.axis_index("c")
        row_base = wid * n_chunks
        base = wid * b_per_w
        pltpu.sync_copy(idx_hbm.at[pl.ds(row_base, n_chunks)], idx_v)

        def body(j, carry):
            pltpu.async_copy(table_hbm.at[idx_v.at[j]], rows_v, sem).wait()
            pltpu.sync_copy(rows_v, out_hbm.at[pl.ds(base + j * _CHUNK, _CHUNK)])
            return carry

        lax.fori_loop(0, n_chunks, body, 0)

    return k(weight, idx2d)


def kernel(input, weight):
    B = input.shape[0] * input.shape[1]
    D = weight.shape[1]
    idx2d = input.reshape(B // _CHUNK, _CHUNK)
    out = _gather(weight, idx2d, B, D)
    return out.reshape(input.shape[0], input.shape[1], D)


# speedup vs baseline: 1.4972x; 1.1454x over previous
import functools

import jax
import jax.numpy as jnp
from jax import lax
from jax.experimental import pallas as pl
from jax.experimental.pallas import tpu as pltpu
from jax.experimental.pallas import tpu_sc as plsc

_CHUNK = 128
_NBUF = 8


@functools.partial(jax.jit, static_argnums=(2, 3))
def _gather(weight, idx2d, B, D):
    info = plsc.get_sparse_core_info()
    NW = info.num_cores * info.num_subcores
    b_per_w = B // NW
    n_chunks = b_per_w // _CHUNK
    n_groups = n_chunks // _NBUF
    mesh = plsc.VectorSubcoreMesh(core_axis_name="c", subcore_axis_name="s")

    @functools.partial(
        pl.kernel,
        mesh=mesh,
        compiler_params=pltpu.CompilerParams(use_tc_tiling_on_sc=False),
        out_type=jax.ShapeDtypeStruct((B, D), jnp.float32),
        scratch_types=[
            pltpu.VMEM((n_chunks, _CHUNK), jnp.int32),
            pltpu.VMEM((_NBUF, _CHUNK, D), jnp.float32),
            pltpu.SemaphoreType.DMA((_NBUF,)),
            pltpu.SemaphoreType.DMA((_NBUF,)),
        ],
    )
    def k(table_hbm, idx_hbm, out_hbm, idx_v, rows_v, gsem, ssem):
        wid = lax.axis_index("s") * info.num_cores + lax.axis_index("c")
        row_base = wid * n_chunks
        base = wid * b_per_w
        pltpu.sync_copy(idx_hbm.at[pl.ds(row_base, n_chunks)], idx_v)

        def start_gather(j, b):
            pltpu.async_copy(table_hbm.at[idx_v.at[j]], rows_v.at[b], gsem.at[b])

        for b in range(_NBUF):
            start_gather(b, b)

        def grp(g, carry):
            for b in range(_NBUF):
                j = g * _NBUF + b
                pltpu.make_async_copy(
                    table_hbm.at[idx_v.at[j]], rows_v.at[b], gsem.at[b]
                ).wait()
                pltpu.async_copy(
                    rows_v.at[b],
                    out_hbm.at[pl.ds(base + j * _CHUNK, _CHUNK)],
                    ssem.at[b],
                )
            for b in range(_NBUF):
                pltpu.make_async_copy(
                    rows_v.at[b], out_hbm.at[pl.ds(0, _CHUNK)], ssem.at[b]
                ).wait()

                @pl.when(g + 1 < n_groups)
                def _():
                    start_gather((g + 1) * _NBUF + b, b)

            return carry

        lax.fori_loop(0, n_groups, grp, 0)

    return k(weight, idx2d)


def kernel(input, weight):
    B = input.shape[0] * input.shape[1]
    D = weight.shape[1]
    idx2d = input.reshape(B // _CHUNK, _CHUNK)
    out = _gather(weight, idx2d, B, D)
    return out.reshape(input.shape[0], input.shape[1], D)


# speedup vs baseline: 1.5743x; 1.0516x over previous
import functools

import jax
import jax.numpy as jnp
from jax import lax
from jax.experimental import pallas as pl
from jax.experimental.pallas import tpu as pltpu
from jax.experimental.pallas import tpu_sc as plsc

_CHUNK = 128
_NBUF = 8


@functools.partial(jax.jit, static_argnums=(2, 3))
def _gather(weight, idx2d, B, D):
    info = plsc.get_sparse_core_info()
    NW = info.num_cores * info.num_subcores
    b_per_w = B // NW
    n_chunks = b_per_w // _CHUNK
    n_groups = n_chunks // _NBUF
    mesh = plsc.VectorSubcoreMesh(core_axis_name="c", subcore_axis_name="s")

    @functools.partial(
        pl.kernel,
        mesh=mesh,
        compiler_params=pltpu.CompilerParams(use_tc_tiling_on_sc=False),
        out_type=jax.ShapeDtypeStruct((B, D), jnp.float32),
        scratch_types=[
            pltpu.VMEM((n_chunks, _CHUNK), jnp.int32),
            pltpu.VMEM((_NBUF, _CHUNK, D), jnp.float32),
            pltpu.SemaphoreType.DMA((_NBUF,)),
            pltpu.SemaphoreType.DMA((_NBUF,)),
        ],
    )
    def k(table_hbm, idx_hbm, out_hbm, idx_v, rows_v, gsem, ssem):
        wid = lax.axis_index("s") * info.num_cores + lax.axis_index("c")
        row_base = wid * n_chunks
        base = wid * b_per_w
        pltpu.sync_copy(idx_hbm.at[pl.ds(row_base, n_chunks)], idx_v)

        def start_gather(j, b):
            pltpu.async_copy(table_hbm.at[idx_v.at[j]], rows_v.at[b], gsem.at[b])

        for b in range(_NBUF):
            start_gather(b, b)

        def grp(g, carry):
            for b in range(_NBUF):
                j = g * _NBUF + b
                pltpu.make_async_copy(
                    table_hbm.at[idx_v.at[j]], rows_v.at[b], gsem.at[b]
                ).wait()
                pltpu.async_copy(
                    rows_v.at[b],
                    out_hbm.at[pl.ds(base + j * _CHUNK, _CHUNK)],
                    ssem.at[b],
                )
            for b in range(_NBUF):
                pltpu.make_async_copy(
                    rows_v.at[b], out_hbm.at[pl.ds(0, _CHUNK)], ssem.at[b]
                ).wait()

                @pl.when(g + 1 < n_groups)
                def _():
                    start_gather((g + 1) * _NBUF + b, b)

            return carry

        lax.fori_loop(0, n_groups, grp, 0)

    return k(weight, idx2d)


def kernel(input, weight):
    BA, SQ = input.shape
    B = BA * SQ
    D = weight.shape[1]
    idx2d = input.T.reshape(B // _CHUNK, _CHUNK)
    out = _gather(weight, idx2d, B, D)
    return out.reshape(SQ, BA, D).transpose(1, 0, 2)


# speedup vs baseline: 1.5768x; 1.0016x over previous
import functools

import jax
import jax.numpy as jnp
from jax import lax
from jax.experimental import pallas as pl
from jax.experimental.pallas import tpu as pltpu
from jax.experimental.pallas import tpu_sc as plsc

_LANE = 128
_SUB = 8


@functools.partial(jax.jit, static_argnums=(2, 3, 4))
def _gather(weight, idx4d, B, D, SQ):
    info = plsc.get_sparse_core_info()
    NC = info.num_cores
    NW = NC * info.num_subcores
    BA = B // SQ
    n_si = SQ // _SUB
    mesh = plsc.VectorSubcoreMesh(core_axis_name="c", subcore_axis_name="s")

    @functools.partial(
        pl.kernel,
        mesh=mesh,
        compiler_params=pltpu.CompilerParams(use_tc_tiling_on_sc=False),
        out_type=jax.ShapeDtypeStruct((SQ, B // SQ, D), jnp.float32),
        scratch_types=[
            pltpu.VMEM((n_si, _SUB, _LANE), jnp.int32),
            pltpu.VMEM((_SUB, _LANE, D), jnp.float32),
            pltpu.SemaphoreType.DMA((_SUB,)),
            pltpu.SemaphoreType.DMA((_SUB,)),
        ],
    )
    def k(table_hbm, idx_hbm, out_hbm, idx_v, rows_v, gsem, ssem):
        w = lax.axis_index("s") * NC + lax.axis_index("c")
        pltpu.sync_copy(idx_hbm.at[:, w], idx_v)

        def start_gather(si, p):
            pltpu.async_copy(
                table_hbm.at[idx_v.at[si, p]], rows_v.at[p], gsem.at[p]
            )

        for p in range(_SUB):
            start_gather(0, p)

        def grp(si, carry):
            for p in range(_SUB):
                pltpu.make_async_copy(
                    table_hbm.at[idx_v.at[si, p]], rows_v.at[p], gsem.at[p]
                ).wait()
                pltpu.async_copy(
                    rows_v.at[p],
                    out_hbm.at[si * _SUB + p, pl.ds(w * _LANE, _LANE)],
                    ssem.at[p],
                )
            for p in range(_SUB):
                pltpu.make_async_copy(
                    rows_v.at[p], out_hbm.at[0, pl.ds(0, _LANE)], ssem.at[p]
                ).wait()

                @pl.when(si + 1 < n_si)
                def _():
                    start_gather(si + 1, p)

            return carry

        lax.fori_loop(0, n_si, grp, 0)

    return k(weight, idx4d)


def kernel(input, weight):
    BA, SQ = input.shape
    B = BA * SQ
    D = weight.shape[1]
    idx4d = (
        input.T.reshape(SQ // _SUB, _SUB, BA // _LANE, _LANE)
        .transpose(0, 2, 1, 3)
    )
    out = _gather(weight, idx4d, B, D, SQ)
    return out.transpose(1, 0, 2)
